# baseline (speedup 1.0000x reference)
import dataclasses

import jax
import jax.numpy as jnp
from jax import lax
from jax.experimental import pallas as pl
from jax.experimental.pallas import tpu as pltpu
from jax.experimental.pallas import tpu_sc as plsc

B, C, D = 1024, 64, 1024
BB = 64
F = 64
NW = 8
NB = F // NW
NSLAB = D // 128
NSTEP = NB * NSLAB
BIG = jnp.int32(1 << 20)


def _tc_body(x_ref, logits_ref, latent_ref):
    xb = x_ref[...]
    sq = jnp.sum(xb * xb, axis=1)
    logits_ref[...] = jnp.sqrt(sq)
    m = jnp.max(sq, axis=1, keepdims=True)
    d_iota = lax.broadcasted_iota(jnp.int32, (BB, D), 1)
    idx = jnp.min(jnp.where(sq == m, d_iota, jnp.int32(D)), axis=1)
    onehot = (d_iota == idx[:, None]).astype(jnp.float32)
    latent_ref[...] = jnp.sum(xb * onehot[:, None, :], axis=2)


def _sqrt16(s):
    i = lax.bitcast_convert_type(s, jnp.int32)
    r = lax.bitcast_convert_type(jnp.int32(0x5F3759DF) - (i >> 1),
                                 jnp.float32)
    for _ in range(3):
        r = r * (1.5 - 0.5 * s * r * r)
    return s * r


def _sc_body(x_hbm, logits_hbm, lat_hbm, buf0, buf1, vsmax, vsidx, lrows,
             lats, best, sem0, sem1):
    w = lax.axis_index("s") * 2 + lax.axis_index("c")
    bs = (B - F) + w * NB
    i16 = lax.iota(jnp.int32, 16)
    active = w < NW

    def slab_copy(s, buf, sem):
        b = bs + s // NSLAB
        off = pl.multiple_of((s % NSLAB) * 128, 128)
        return pltpu.make_async_copy(
            x_hbm.at[b, :, pl.ds(off, 128)], buf, sem
        )

    @pl.when(active)
    def _():
        slab_copy(0, buf0, sem0).start()
        slab_copy(1, buf1, sem1).start()

    def do_slab(s, buf, sem):
        row8 = s // NSLAB
        k = s % NSLAB

        @pl.when(k == 0)
        def _():
            best[0] = -1.0

        slab_copy(s, buf, sem).wait()

        def acc_step(r, accs):
            return tuple(
                accs[j] + buf[r, pl.ds(16 * j, 16)] * buf[r, pl.ds(16 * j, 16)]
                for j in range(8)
            )

        accs = lax.fori_loop(
            0, C, acc_step, tuple(jnp.zeros((16,), jnp.float32)
                                  for _ in range(8))
        )

        vsmax[...] = accs[0]
        vsidx[...] = k * 128 + i16
        lrows[row8, pl.ds(k * 128, 16)] = _sqrt16(accs[0])
        for j in range(1, 8):
            d_vec = k * 128 + 16 * j + i16
            gt = accs[j] > vsmax[...]
            vsidx[...] = jnp.where(gt, d_vec, vsidx[...])
            vsmax[...] = jnp.where(gt, accs[j], vsmax[...])
            lrows[row8, pl.ds(k * 128 + 16 * j, 16)] = _sqrt16(accs[j])

        m_k = lax.reduce_max(vsmax[...], axes=(0,))

        @pl.when(m_k > best[0])
        def _():
            best[0] = m_k
            dmin = lax.reduce_min(
                jnp.where(vsmax[...] == m_k, vsidx[...], BIG), axes=(0,)
            )
            col = i16 * 0 + dmin % 128
            for t in range(4):
                lats[row8, pl.ds(16 * t, 16)] = plsc.load_gather(
                    buf, [16 * t + i16, col]
                )

        @pl.when(s + 2 < NSTEP)
        def _():
            slab_copy(s + 2, buf, sem).start()

    @pl.when(active)
    def _():
        @pl.loop(0, NSTEP, step=2)
        def _(s):
            do_slab(s, buf0, sem0)
            do_slab(s + 1, buf1, sem1)

        out_r = w * NB
        pltpu.sync_copy(lrows, logits_hbm.at[pl.ds(out_r, NB), :])
        pltpu.sync_copy(lats, lat_hbm.at[pl.ds(out_r, NB), :])


def _sc_part(x):
    cp = pltpu.CompilerParams()
    if "needs_layout_passes" in pltpu.CompilerParams.__dataclass_fields__:
        cp = dataclasses.replace(cp, needs_layout_passes=False)
    mesh = plsc.VectorSubcoreMesh(core_axis_name="c", subcore_axis_name="s")
    kern = pl.kernel(
        _sc_body,
        out_type=[
            jax.ShapeDtypeStruct((F, D), jnp.float32),
            jax.ShapeDtypeStruct((F, C), jnp.float32),
        ],
        mesh=mesh,
        scratch_types=[
            pltpu.VMEM((C, 128), jnp.float32),
            pltpu.VMEM((C, 128), jnp.float32),
            pltpu.VMEM((16,), jnp.float32),
            pltpu.VMEM((16,), jnp.int32),
            pltpu.VMEM((NB, D), jnp.float32),
            pltpu.VMEM((NB, C), jnp.float32),
            pltpu.SMEM((1,), jnp.float32),
            pltpu.SemaphoreType.DMA,
            pltpu.SemaphoreType.DMA,
        ],
        compiler_params=cp,
    )
    return kern(x)


@jax.jit
def kernel(x):
    logits_sc, latent_sc = _sc_part(x)
    logits_tc, latent_tc = pl.pallas_call(
        _tc_body,
        grid=((B - F) // BB,),
        in_specs=[pl.BlockSpec((BB, C, D), lambda i: (i, 0, 0))],
        out_specs=[
            pl.BlockSpec((BB, D), lambda i: (i, 0)),
            pl.BlockSpec((BB, C), lambda i: (i, 0)),
        ],
        out_shape=[
            jax.ShapeDtypeStruct((B - F, D), jnp.float32),
            jax.ShapeDtypeStruct((B - F, C), jnp.float32),
        ],
        compiler_params=pltpu.CompilerParams(
            dimension_semantics=("parallel",)
        ),
    )(x)
    logits = jnp.concatenate([logits_tc, logits_sc], axis=0)
    latent = jnp.concatenate([latent_tc, latent_sc], axis=0)
    return (logits, latent)


# speedup vs baseline: 1.1913x; 1.1913x over previous
import jax
import jax.numpy as jnp
from jax import lax
from jax.experimental import pallas as pl
from jax.experimental.pallas import tpu as pltpu

B, C, D = 1024, 64, 1024
BB = 64


def _fused_body(x_ref, logits_ref, latent_ref):
    xb = x_ref[...]
    sq = jnp.sum(xb * xb, axis=1)
    logits_ref[...] = jnp.sqrt(sq)
    m = jnp.max(sq, axis=1, keepdims=True)
    d_iota = lax.broadcasted_iota(jnp.int32, (BB, D), 1)
    idx = jnp.min(jnp.where(sq == m, d_iota, jnp.int32(D)), axis=1)
    onehot = (d_iota == idx[:, None]).astype(jnp.float32)
    latent_ref[...] = jnp.sum(xb * onehot[:, None, :], axis=2)


@jax.jit
def kernel(x):
    logits, latent = pl.pallas_call(
        _fused_body,
        grid=(B // BB,),
        in_specs=[pl.BlockSpec((BB, C, D), lambda i: (i, 0, 0))],
        out_specs=[
            pl.BlockSpec((BB, D), lambda i: (i, 0)),
            pl.BlockSpec((BB, C), lambda i: (i, 0)),
        ],
        out_shape=[
            jax.ShapeDtypeStruct((B, D), jnp.float32),
            jax.ShapeDtypeStruct((B, C), jnp.float32),
        ],
        compiler_params=pltpu.CompilerParams(
            dimension_semantics=("parallel",)
        ),
    )(x)
    return (logits, latent)
